# baseline (speedup 1.0000x reference)
import dataclasses
import math

import jax
import jax.numpy as jnp
from jax import lax
from jax.experimental import pallas as pl
from jax.experimental.pallas import tpu as pltpu
from jax.experimental.pallas import tpu_sc as plsc

D_MODEL = 32
LANES = 16
SCALE = math.sqrt(D_MODEL)
NW = 32
BATCH = 4096
HIST = 200
NBT = BATCH // 128
UNITS = HIST * NBT
UPW = UNITS // NW


def _compiler_params():
    cp = pltpu.CompilerParams(use_tc_tiling_on_sc=False)
    if "needs_layout_passes" in pltpu.CompilerParams.__dataclass_fields__:
        cp = dataclasses.replace(cp, needs_layout_passes=False)
    return cp


def _sc_gather_scale(xt, lut):
    mesh = plsc.VectorSubcoreMesh(core_axis_name="c", subcore_axis_name="s")

    @pl.kernel(
        out_type=jax.ShapeDtypeStruct((HIST, 4, NBT, 8, 128), jnp.float32),
        mesh=mesh,
        scratch_types=[
            pltpu.VMEM((128,), jnp.int32),
            pltpu.VMEM((128,), jnp.int32),
            pltpu.VMEM((128, D_MODEL), jnp.float32),
            pltpu.VMEM((128, D_MODEL), jnp.float32),
            pltpu.VMEM((4, 8, 128), jnp.float32),
            pltpu.VMEM((4, 8, 128), jnp.float32),
            pltpu.SemaphoreType.DMA,
            pltpu.SemaphoreType.DMA,
            pltpu.SemaphoreType.DMA,
            pltpu.SemaphoreType.DMA,
        ],
        compiler_params=_compiler_params(),
    )
    def kernel_fn(lut_hbm, xt_hbm, out_hbm, i0, i1, r0, r1, s0, s1,
                  g0, g1, o0, o1):
        idx = (i0, i1)
        rows = (r0, r1)
        stg = (s0, s1)
        gsem = (g0, g1)
        osem = (o0, o1)
        wid = lax.axis_index("s") * 2 + lax.axis_index("c")
        u0 = wid * UPW
        iota = lax.iota(jnp.int32, LANES)
        rowvs = [iota + LANES * k for k in range(128 // LANES)]

        def load_idx(u, p):
            pltpu.sync_copy(
                xt_hbm.at[u // NBT, pl.ds((u % NBT) * 128, 128)], idx[p]
            )

        def fire_gather(p):
            pltpu.async_copy(lut_hbm.at[idx[p]], rows[p], gsem[p])

        def wait_gather(p):
            pltpu.make_async_copy(lut_hbm.at[idx[p]], rows[p], gsem[p]).wait()

        def fire_out(u, p):
            for dt in range(4):
                pltpu.async_copy(
                    stg[p].at[dt], out_hbm.at[u // NBT, dt, u % NBT], osem[p]
                )

        def wait_out(u, p):
            for dt in range(4):
                pltpu.make_async_copy(
                    stg[p].at[dt], out_hbm.at[u // NBT, dt, u % NBT], osem[p]
                ).wait()

        load_idx(u0, 0)
        fire_gather(0)

        @pl.loop(0, UPW, step=2)
        def _(t):
            for p in range(2):
                tt = t + p
                u = u0 + tt

                @pl.when(tt + 1 < UPW)
                def _():
                    load_idx(u + 1, 1 - p)
                    fire_gather(1 - p)

                wait_gather(p)

                @pl.when(tt >= 2)
                def _():
                    wait_out(u - 2, p)

                for dt in range(4):
                    for ds in range(8):
                        colv = jnp.full((LANES,), dt * 8 + ds, jnp.int32)
                        for k in range(128 // LANES):
                            val = plsc.load_gather(rows[p], [rowvs[k], colv])
                            stg[p].at[dt, ds, pl.ds(LANES * k, LANES)][
                                ...
                            ] = val * SCALE

                fire_out(u, p)

        wait_out(u0 + UPW - 2, 0)
        wait_out(u0 + UPW - 1, 1)

    return kernel_fn(lut, xt)


@jax.jit
def kernel(x, lut):
    xt = x.astype(jnp.int32).T
    out_lin = _sc_gather_scale(xt, lut)
    return out_lin.transpose(2, 4, 0, 1, 3).reshape(BATCH, HIST, D_MODEL)


# speedup vs baseline: 1.0486x; 1.0486x over previous
import dataclasses
import math

import jax
import jax.numpy as jnp
from jax import lax
from jax.experimental import pallas as pl
from jax.experimental.pallas import tpu as pltpu
from jax.experimental.pallas import tpu_sc as plsc

D_MODEL = 32
LANES = 16
SCALE = math.sqrt(D_MODEL)
NW = 32
BATCH = 4096
HIST = 200
NBT = BATCH // 128
UNITS = HIST * NBT
UPW = UNITS // NW
IPW = UPW * 128


def _compiler_params():
    cp = pltpu.CompilerParams(use_tc_tiling_on_sc=False)
    if "needs_layout_passes" in pltpu.CompilerParams.__dataclass_fields__:
        cp = dataclasses.replace(cp, needs_layout_passes=False)
    return cp


def _sc_gather_scale(xf, lut):
    mesh = plsc.VectorSubcoreMesh(core_axis_name="c", subcore_axis_name="s")

    @pl.kernel(
        out_type=jax.ShapeDtypeStruct((HIST, 4, NBT, 8, 128), jnp.float32),
        mesh=mesh,
        scratch_types=[
            pltpu.VMEM((IPW,), jnp.int32),
            pltpu.VMEM((128, D_MODEL), jnp.float32),
            pltpu.VMEM((128, D_MODEL), jnp.float32),
            pltpu.VMEM((128, D_MODEL), jnp.float32),
            pltpu.VMEM((128, D_MODEL), jnp.float32),
            pltpu.VMEM((4, 8, 128), jnp.float32),
            pltpu.VMEM((4, 8, 128), jnp.float32),
            pltpu.SemaphoreType.DMA,
            pltpu.SemaphoreType.DMA,
            pltpu.SemaphoreType.DMA,
            pltpu.SemaphoreType.DMA,
            pltpu.SemaphoreType.DMA,
            pltpu.SemaphoreType.DMA,
        ],
        compiler_params=_compiler_params(),
    )
    def kernel_fn(lut_hbm, xf_hbm, out_hbm, slab, r0, r1, r2, r3, s0, s1,
                  g0, g1, g2, g3, o0, o1):
        rows = (r0, r1, r2, r3)
        stg = (s0, s1)
        gsem = (g0, g1, g2, g3)
        osem = (o0, o1)
        wid = lax.axis_index("s") * 2 + lax.axis_index("c")
        u0 = wid * UPW
        iota = lax.iota(jnp.int32, LANES)
        rowvs = [iota + LANES * k for k in range(128 // LANES)]

        def gather_copy(t, j):
            return pltpu.make_async_copy(
                lut_hbm.at[slab.at[pl.ds(t * 128, 128)]], rows[j], gsem[j]
            )

        def out_copy(u, dt, p):
            return pltpu.make_async_copy(
                stg[p].at[dt], out_hbm.at[u // NBT, dt, u % NBT], osem[p]
            )

        pltpu.sync_copy(xf_hbm.at[pl.ds(u0 * 128, IPW)], slab)
        for j in range(4):
            gather_copy(j, j).start()

        @pl.loop(0, UPW, step=4)
        def _(t):
            for j in range(4):
                tt = t + j
                u = u0 + tt
                p = j % 2

                gather_copy(tt, j).wait()

                @pl.when(tt >= 2)
                def _():
                    for dt in range(4):
                        out_copy(u - 2, dt, p).wait()

                for dt in range(4):
                    for ds in range(8):
                        colv = jnp.full((LANES,), dt * 8 + ds, jnp.int32)
                        for k in range(128 // LANES):
                            val = plsc.load_gather(rows[j], [rowvs[k], colv])
                            stg[p].at[dt, ds, pl.ds(LANES * k, LANES)][
                                ...
                            ] = val * SCALE

                @pl.when(tt + 4 < UPW)
                def _():
                    gather_copy(tt + 4, j).start()

                for dt in range(4):
                    out_copy(u, dt, p).start()

        for dt in range(4):
            out_copy(u0 + UPW - 2, dt, 0).wait()
        for dt in range(4):
            out_copy(u0 + UPW - 1, dt, 1).wait()

    return kernel_fn(lut, xf)


@jax.jit
def kernel(x, lut):
    xf = x.astype(jnp.int32).T.reshape(BATCH * HIST)
    out_lin = _sc_gather_scale(xf, lut)
    return out_lin.transpose(2, 4, 0, 1, 3).reshape(BATCH, HIST, D_MODEL)


# speedup vs baseline: 1.2729x; 1.2139x over previous
import dataclasses
import math

import jax
import jax.numpy as jnp
from jax import lax
from jax.experimental import pallas as pl
from jax.experimental.pallas import tpu as pltpu
from jax.experimental.pallas import tpu_sc as plsc

D_MODEL = 32
LANES = 16
SCALE = math.sqrt(D_MODEL)
NW = 32
BATCH = 4096
HIST = 200
NBT = BATCH // 128
UNITS = HIST * NBT
UPW = UNITS // NW
IPW = UPW * 128


def _compiler_params():
    cp = pltpu.CompilerParams(use_tc_tiling_on_sc=False)
    if "needs_layout_passes" in pltpu.CompilerParams.__dataclass_fields__:
        cp = dataclasses.replace(cp, needs_layout_passes=False)
    return cp


def _sc_gather_scale(xf, lut):
    mesh = plsc.VectorSubcoreMesh(core_axis_name="c", subcore_axis_name="s")

    @pl.kernel(
        out_type=jax.ShapeDtypeStruct((HIST, 4, NBT, 1024), jnp.float32),
        mesh=mesh,
        scratch_types=[
            pltpu.VMEM((IPW,), jnp.int32),
            pltpu.VMEM((128, D_MODEL), jnp.float32),
            pltpu.VMEM((128, D_MODEL), jnp.float32),
            pltpu.VMEM((128, D_MODEL), jnp.float32),
            pltpu.VMEM((128, D_MODEL), jnp.float32),
            pltpu.VMEM((4096,), jnp.float32),
            pltpu.VMEM((4096,), jnp.float32),
            pltpu.SemaphoreType.DMA,
            pltpu.SemaphoreType.DMA,
            pltpu.SemaphoreType.DMA,
            pltpu.SemaphoreType.DMA,
            pltpu.SemaphoreType.DMA,
            pltpu.SemaphoreType.DMA,
        ],
        compiler_params=_compiler_params(),
    )
    def kernel_fn(lut_hbm, xf_hbm, out_hbm, slab, r0, r1, r2, r3, s0, s1,
                  g0, g1, g2, g3, o0, o1):
        rows = (r0, r1, r2, r3)
        stg = (s0, s1)
        gsem = (g0, g1, g2, g3)
        osem = (o0, o1)
        wid = lax.axis_index("s") * 2 + lax.axis_index("c")
        u0 = wid * UPW
        iota = lax.iota(jnp.int32, LANES)
        cbase = ((iota >> 3) << 10) + ((iota & 7) << 7)

        def gather_copy(t, j):
            return pltpu.make_async_copy(
                lut_hbm.at[slab.at[pl.ds(t * 128, 128)]], rows[j], gsem[j]
            )

        def out_copy(u, dt, p):
            return pltpu.make_async_copy(
                stg[p].at[pl.ds(dt * 1024, 1024)],
                out_hbm.at[u // NBT, dt, u % NBT],
                osem[p],
            )

        pltpu.sync_copy(xf_hbm.at[pl.ds(u0 * 128, IPW)], slab)
        for j in range(4):
            gather_copy(j, j).start()

        @pl.loop(0, UPW, step=4)
        def _(t):
            for j in range(4):
                tt = t + j
                u = u0 + tt
                p = j % 2

                gather_copy(tt, j).wait()

                @pl.when(tt >= 2)
                def _():
                    for dt in range(4):
                        out_copy(u - 2, dt, p).wait()

                @pl.loop(0, 128, step=4)
                def _(bl):
                    for q in range(4):
                        b = bl + q
                        dst = cbase + b
                        v0 = rows[j].at[b, pl.ds(0, LANES)][...] * SCALE
                        plsc.store_scatter(stg[p], [dst], v0)
                        v1 = rows[j].at[b, pl.ds(LANES, LANES)][...] * SCALE
                        plsc.store_scatter(stg[p], [dst + 2048], v1)

                @pl.when(tt + 4 < UPW)
                def _():
                    gather_copy(tt + 4, j).start()

                for dt in range(4):
                    out_copy(u, dt, p).start()

        for dt in range(4):
            out_copy(u0 + UPW - 2, dt, 0).wait()
        for dt in range(4):
            out_copy(u0 + UPW - 1, dt, 1).wait()

    return kernel_fn(lut, xf)


@jax.jit
def kernel(x, lut):
    xf = x.astype(jnp.int32).T.reshape(BATCH * HIST)
    out_lin = _sc_gather_scale(xf, lut)
    return (
        out_lin.reshape(HIST, 4, NBT, 8, 128)
        .transpose(2, 4, 0, 1, 3)
        .reshape(BATCH, HIST, D_MODEL)
    )


# speedup vs baseline: 1.7673x; 1.3884x over previous
import dataclasses
import math

import jax
import jax.numpy as jnp
from jax import lax
from jax.experimental import pallas as pl
from jax.experimental.pallas import tpu as pltpu
from jax.experimental.pallas import tpu_sc as plsc

D_MODEL = 32
LANES = 16
SCALE = math.sqrt(D_MODEL)
NW = 32
BATCH = 4096
HIST = 200
NBT = BATCH // 128
UNITS = HIST * NBT
UPW = UNITS // NW
IPW = UPW * 128


def _compiler_params():
    cp = pltpu.CompilerParams(use_tc_tiling_on_sc=False)
    if "needs_layout_passes" in pltpu.CompilerParams.__dataclass_fields__:
        cp = dataclasses.replace(cp, needs_layout_passes=False)
    return cp


def _tc_prescale_linearize(lut_t):
    c_blk = 8192
    n_blk = -(-1000000 // c_blk)

    def body(in_ref, out_ref):
        for a in range(4):
            out_ref[:, a * 32:(a + 1) * 32] = (
                in_ref[:, a * 2048:(a + 1) * 2048] * SCALE
            ).T

    return pl.pallas_call(
        body,
        grid=(n_blk,),
        in_specs=[pl.BlockSpec((32, c_blk), lambda i: (0, i))],
        out_specs=pl.BlockSpec((c_blk // 4, 128), lambda i: (i, 0)),
        out_shape=jax.ShapeDtypeStruct((n_blk * c_blk // 4, 128), jnp.float32),
    )(lut_t)


def _sc_gather_scale(xf, lut):
    mesh = plsc.VectorSubcoreMesh(core_axis_name="c", subcore_axis_name="s")

    @pl.kernel(
        out_type=jax.ShapeDtypeStruct((HIST, 4, NBT, 1024), jnp.float32),
        mesh=mesh,
        scratch_types=[
            pltpu.VMEM((IPW,), jnp.int32),
            pltpu.VMEM((128, D_MODEL), jnp.float32),
            pltpu.VMEM((128, D_MODEL), jnp.float32),
            pltpu.VMEM((128, D_MODEL), jnp.float32),
            pltpu.VMEM((128, D_MODEL), jnp.float32),
            pltpu.VMEM((4096,), jnp.float32),
            pltpu.VMEM((4096,), jnp.float32),
            pltpu.SemaphoreType.DMA,
            pltpu.SemaphoreType.DMA,
            pltpu.SemaphoreType.DMA,
            pltpu.SemaphoreType.DMA,
            pltpu.SemaphoreType.DMA,
            pltpu.SemaphoreType.DMA,
        ],
        compiler_params=_compiler_params(),
    )
    def kernel_fn(lut_hbm, xf_hbm, out_hbm, slab, r0, r1, r2, r3, s0, s1,
                  g0, g1, g2, g3, o0, o1):
        rows = (r0, r1, r2, r3)
        stg = (s0, s1)
        gsem = (g0, g1, g2, g3)
        osem = (o0, o1)
        wid = lax.axis_index("s") * 2 + lax.axis_index("c")
        u0 = wid * UPW
        iota = lax.iota(jnp.int32, LANES)
        cbase = ((iota >> 3) << 10) + ((iota & 7) << 7)

        def gather_copy(t, j):
            return pltpu.make_async_copy(
                lut_hbm.at[slab.at[pl.ds(t * 128, 128)]], rows[j], gsem[j]
            )

        def out_copy(u, dt, p):
            return pltpu.make_async_copy(
                stg[p].at[pl.ds(dt * 1024, 1024)],
                out_hbm.at[u // NBT, dt, u % NBT],
                osem[p],
            )

        pltpu.sync_copy(xf_hbm.at[pl.ds(u0 * 128, IPW)], slab)

        @pl.loop(0, IPW, step=64)
        def _(s):
            for q in range(4):
                off = s + LANES * q
                v = slab.at[pl.ds(off, LANES)][...]
                j2 = (v & -8192) + ((v & 2047) << 2) + ((v >> 11) & 3)
                slab.at[pl.ds(off, LANES)][...] = j2

        for j in range(4):
            gather_copy(j, j).start()

        @pl.loop(0, UPW, step=4)
        def _(t):
            for j in range(4):
                tt = t + j
                u = u0 + tt
                p = j % 2

                gather_copy(tt, j).wait()

                @pl.when(tt >= 2)
                def _():
                    for dt in range(4):
                        out_copy(u - 2, dt, p).wait()

                @pl.loop(0, 128, step=4)
                def _(bl):
                    for q in range(4):
                        b = bl + q
                        dst = cbase + b
                        v0 = rows[j].at[b, pl.ds(0, LANES)][...]
                        plsc.store_scatter(stg[p], [dst], v0)
                        v1 = rows[j].at[b, pl.ds(LANES, LANES)][...]
                        plsc.store_scatter(stg[p], [dst + 2048], v1)

                @pl.when(tt + 4 < UPW)
                def _():
                    gather_copy(tt + 4, j).start()

                for dt in range(4):
                    out_copy(u, dt, p).start()

        for dt in range(4):
            out_copy(u0 + UPW - 2, dt, 0).wait()
        for dt in range(4):
            out_copy(u0 + UPW - 1, dt, 1).wait()

    return kernel_fn(lut, xf)


@jax.jit
def kernel(x, lut):
    xf = x.astype(jnp.int32).T.reshape(BATCH * HIST)
    lut_lin = _tc_prescale_linearize(lut.T).reshape(-1, D_MODEL)
    out_lin = _sc_gather_scale(xf, lut_lin)
    return (
        out_lin.reshape(HIST, 4, NBT, 8, 128)
        .transpose(2, 4, 0, 1, 3)
        .reshape(BATCH, HIST, D_MODEL)
    )
